# baseline (speedup 1.0000x reference)
import functools

import jax
import jax.numpy as jnp
from jax import lax
from jax.experimental import pallas as pl
from jax.experimental.pallas import tpu as pltpu
from jax.experimental.pallas import tpu_sc as plsc

T = 64
H = 1024
F = 512
E = 64
ZE = 16
K = 2
SCALE = 2.5
NC = E + ZE


def _logits_kernel(x_ref, wr_ref, bias_ref, s_ref, sc_ref):
    logitsT = lax.dot_general(wr_ref[...], x_ref[...], (((1,), (1,)), ((), ())),
                              preferred_element_type=jnp.float32)
    sT = jax.nn.softmax(logitsT, axis=0)
    s_ref[...] = sT
    sc_ref[...] = sT + bias_ref[...]


def _sc_router(s_hbm, sc_hbm, i1_hbm, i2_hbm, w1_hbm, w2_hbm, zsc_hbm,
               s_v, sc_v, i1_v, i2_v, w1_v, w2_v, zsc_v):
    first = (lax.axis_index("s") == 0) & (lax.axis_index("c") == 0)

    @pl.when(first)
    def _():
        pltpu.sync_copy(s_hbm, s_v)
        pltpu.sync_copy(sc_hbm, sc_v)

        for g in range(4):
            def body(c, carry):
                m1, i1, w1, m2, i2, w2 = carry
                vc = sc_v[c, pl.ds(g * 16, 16)]
                vs = s_v[c, pl.ds(g * 16, 16)]
                cvec = jnp.full((16,), 0, jnp.int32) + c
                gt1 = vc > m1
                gt2 = vc > m2
                m2n = jnp.where(gt1, m1, jnp.where(gt2, vc, m2))
                i2n = jnp.where(gt1, i1, jnp.where(gt2, cvec, i2))
                w2n = jnp.where(gt1, w1, jnp.where(gt2, vs, w2))
                m1n = jnp.where(gt1, vc, m1)
                i1n = jnp.where(gt1, cvec, i1)
                w1n = jnp.where(gt1, vs, w1)
                return m1n, i1n, w1n, m2n, i2n, w2n

            neg = jnp.full((16,), -jnp.inf, jnp.float32)
            zi = jnp.zeros((16,), jnp.int32)
            zf = jnp.zeros((16,), jnp.float32)
            m1, i1, w1, m2, i2, w2 = lax.fori_loop(
                0, NC, body, (neg, zi, zf, neg, zi, zf))

            w1s = w1 * SCALE
            w2s = w2 * SCALE
            z1 = i1 >= E
            z2 = i2 >= E
            zsc = jnp.where(z1, w1s, 0.0) + jnp.where(z2, w2s, 0.0)
            w1f = jnp.where(z1, 0.0, w1s)
            w2f = jnp.where(z2, 0.0, w2s)

            sl = pl.ds(g * 16, 16)
            i1_v[sl] = i1
            i2_v[sl] = i2
            w1_v[sl] = w1f
            w2_v[sl] = w2f
            zsc_v[sl] = zsc

        pltpu.sync_copy(i1_v, i1_hbm)
        pltpu.sync_copy(i2_v, i2_hbm)
        pltpu.sync_copy(w1_v, w1_hbm)
        pltpu.sync_copy(w2_v, w2_hbm)
        pltpu.sync_copy(zsc_v, zsc_hbm)


def _sched_kernel(i1_ref, i2_ref, sched_ref):
    i1 = i1_ref[...]
    i2 = i2_ref[...]
    eio = lax.broadcasted_iota(jnp.int32, (T, E), 1)
    hitf = ((i1 == eio) | (i2 == eio)).astype(jnp.float32)
    ones_col = jnp.ones((T, 1), jnp.float32)
    nhit = lax.dot_general(hitf, ones_col, (((0,), (0,)), ((), ())),
                           preferred_element_type=jnp.float32)
    activef = (nhit > 0.0).astype(jnp.float32)
    io0 = lax.broadcasted_iota(jnp.int32, (E, E), 0)
    io1 = lax.broadcasted_iota(jnp.int32, (E, E), 1)
    ltri = (io0 >= io1).astype(jnp.float32)
    rank = lax.dot_general(ltri, activef, (((1,), (0,)), ((), ())),
                           preferred_element_type=jnp.float32)
    jrow = lax.broadcasted_iota(jnp.int32, (1, E), 1)
    onehot = ((rank.astype(jnp.int32) == jrow + 1) &
              (activef > 0.0)).astype(jnp.float32)
    erow = lax.broadcasted_iota(jnp.int32, (1, E), 1).astype(jnp.float32)
    sched_row = lax.dot_general(erow, onehot, (((1,), (0,)), ((), ())),
                                preferred_element_type=jnp.float32)
    count = jnp.sum(activef, axis=0, keepdims=True).astype(jnp.int32)
    ecol = lax.broadcasted_iota(jnp.int32, (E, 1), 0)
    last = jnp.max(jnp.where(activef > 0.0, ecol, -1), axis=0, keepdims=True)
    sched = jnp.where(jrow < count, sched_row.astype(jnp.int32),
                      jnp.maximum(last, 0))
    sched_ref[...] = jnp.zeros((8, 128), jnp.int32)
    sched_ref[0:1, 0:E] = sched
    sched_ref[0:1, E:E + 1] = count


def _expert_kernel(sched_ref, count_ref, x_ref, i1_ref, i2_ref, w1_ref,
                   w2c_ref, zsc_ref, w13_ref, w2_ref, out_ref):
    i = pl.program_id(0)
    e = sched_ref[i]

    @pl.when(i == 0)
    def _init():
        out_ref[...] = x_ref[...] * zsc_ref[...]

    @pl.when(i < count_ref[0])
    def _expert():
        x = x_ref[...]
        h13 = lax.dot_general(x, w13_ref[0], (((1,), (1,)), ((), ())),
                              preferred_element_type=jnp.float32)
        gate = h13[:, :F]
        up = h13[:, F:]
        act = gate * jax.nn.sigmoid(gate) * up
        oute = lax.dot_general(act, w2_ref[0], (((1,), (1,)), ((), ())),
                               preferred_element_type=jnp.float32)
        wcol = (jnp.where(i1_ref[...] == e, w1_ref[...], 0.0)
                + jnp.where(i2_ref[...] == e, w2c_ref[...], 0.0))
        out_ref[...] += wcol * oute


def kernel(hidden_states, router_weight, e_score_correction_bias, w13_weight, w2_weight):
    biasT = e_score_correction_bias.reshape(NC, 1)
    sT, scT = pl.pallas_call(
        _logits_kernel,
        in_specs=[
            pl.BlockSpec((T, H), lambda: (0, 0)),
            pl.BlockSpec((NC, H), lambda: (0, 0)),
            pl.BlockSpec((NC, 1), lambda: (0, 0)),
        ],
        out_specs=[
            pl.BlockSpec((NC, T), lambda: (0, 0)),
            pl.BlockSpec((NC, T), lambda: (0, 0)),
        ],
        out_shape=[
            jax.ShapeDtypeStruct((NC, T), jnp.float32),
            jax.ShapeDtypeStruct((NC, T), jnp.float32),
        ],
    )(hidden_states, router_weight, biasT)

    sc_router = functools.partial(
        pl.kernel,
        out_type=[
            jax.ShapeDtypeStruct((T,), jnp.int32),
            jax.ShapeDtypeStruct((T,), jnp.int32),
            jax.ShapeDtypeStruct((T,), jnp.float32),
            jax.ShapeDtypeStruct((T,), jnp.float32),
            jax.ShapeDtypeStruct((T,), jnp.float32),
        ],
        mesh=plsc.VectorSubcoreMesh(core_axis_name="c", subcore_axis_name="s"),
        scratch_types=[
            pltpu.VMEM((NC, T), jnp.float32),
            pltpu.VMEM((NC, T), jnp.float32),
            pltpu.VMEM((T,), jnp.int32),
            pltpu.VMEM((T,), jnp.int32),
            pltpu.VMEM((T,), jnp.float32),
            pltpu.VMEM((T,), jnp.float32),
            pltpu.VMEM((T,), jnp.float32),
        ],
    )(_sc_router)
    i1, i2, w1, w2, zsc = sc_router(sT, scT)

    i1c = i1.reshape(T, 1)
    i2c = i2.reshape(T, 1)
    w1c = w1.reshape(T, 1)
    w2c = w2.reshape(T, 1)
    zscc = zsc.reshape(T, 1)

    sched2d = pl.pallas_call(
        _sched_kernel,
        in_specs=[
            pl.BlockSpec((T, 1), lambda: (0, 0)),
            pl.BlockSpec((T, 1), lambda: (0, 0)),
        ],
        out_specs=pl.BlockSpec((8, 128), lambda: (0, 0)),
        out_shape=jax.ShapeDtypeStruct((8, 128), jnp.int32),
    )(i1c, i2c)

    sched = sched2d[0, 0:E]
    count = sched2d[0, E:E + 1]

    grid_spec = pltpu.PrefetchScalarGridSpec(
        num_scalar_prefetch=2,
        grid=(E,),
        in_specs=[
            pl.BlockSpec((T, H), lambda i, s, c: (0, 0)),
            pl.BlockSpec((T, 1), lambda i, s, c: (0, 0)),
            pl.BlockSpec((T, 1), lambda i, s, c: (0, 0)),
            pl.BlockSpec((T, 1), lambda i, s, c: (0, 0)),
            pl.BlockSpec((T, 1), lambda i, s, c: (0, 0)),
            pl.BlockSpec((T, 1), lambda i, s, c: (0, 0)),
            pl.BlockSpec((1, 2 * F, H), lambda i, s, c: (s[i], 0, 0)),
            pl.BlockSpec((1, H, F), lambda i, s, c: (s[i], 0, 0)),
        ],
        out_specs=pl.BlockSpec((T, H), lambda i, s, c: (0, 0)),
    )
    return pl.pallas_call(
        _expert_kernel,
        grid_spec=grid_spec,
        out_shape=jax.ShapeDtypeStruct((T, H), jnp.float32),
    )(sched, count, hidden_states, i1c, i2c, w1c, w2c, zscc, w13_weight, w2_weight)


# speedup vs baseline: 1.0056x; 1.0056x over previous
import functools

import jax
import jax.numpy as jnp
from jax import lax
from jax.experimental import pallas as pl
from jax.experimental.pallas import tpu as pltpu
from jax.experimental.pallas import tpu_sc as plsc

T = 64
H = 1024
F = 512
E = 64
ZE = 16
K = 2
SCALE = 2.5
NC = E + ZE


def _logits_kernel(x_ref, wr_ref, bias_ref, s_ref, sc_ref):
    logitsT = lax.dot_general(wr_ref[...], x_ref[...], (((1,), (1,)), ((), ())),
                              preferred_element_type=jnp.float32)
    sT = jax.nn.softmax(logitsT, axis=0)
    s_ref[...] = sT
    sc_ref[...] = sT + bias_ref[...]


def _sc_router(s_hbm, sc_hbm, i1_hbm, i2_hbm, w1_hbm, w2_hbm, zsc_hbm,
               s_v, sc_v, i1_v, i2_v, w1_v, w2_v, zsc_v):
    wid = lax.axis_index("s") * 2 + lax.axis_index("c")

    @pl.when(wid < 4)
    def _():
        off = pl.multiple_of(wid * 16, 16)
        pltpu.sync_copy(s_hbm, s_v)
        pltpu.sync_copy(sc_hbm, sc_v)

        def body(c, carry):
            m1, i1, w1, m2, i2, w2 = carry
            vc = sc_v[c, pl.ds(off, 16)]
            vs = s_v[c, pl.ds(off, 16)]
            cvec = jnp.full((16,), 0, jnp.int32) + c
            gt1 = vc > m1
            gt2 = vc > m2
            m2n = jnp.where(gt1, m1, jnp.where(gt2, vc, m2))
            i2n = jnp.where(gt1, i1, jnp.where(gt2, cvec, i2))
            w2n = jnp.where(gt1, w1, jnp.where(gt2, vs, w2))
            m1n = jnp.where(gt1, vc, m1)
            i1n = jnp.where(gt1, cvec, i1)
            w1n = jnp.where(gt1, vs, w1)
            return m1n, i1n, w1n, m2n, i2n, w2n

        neg = jnp.full((16,), -jnp.inf, jnp.float32)
        zi = jnp.zeros((16,), jnp.int32)
        zf = jnp.zeros((16,), jnp.float32)
        m1, i1, w1, m2, i2, w2 = lax.fori_loop(
            0, NC, body, (neg, zi, zf, neg, zi, zf))

        w1s = w1 * SCALE
        w2s = w2 * SCALE
        z1 = i1 >= E
        z2 = i2 >= E
        zsc = jnp.where(z1, w1s, 0.0) + jnp.where(z2, w2s, 0.0)
        w1f = jnp.where(z1, 0.0, w1s)
        w2f = jnp.where(z2, 0.0, w2s)

        i1_v[...] = i1
        i2_v[...] = i2
        w1_v[...] = w1f
        w2_v[...] = w2f
        zsc_v[...] = zsc

        sl = pl.ds(off, 16)
        pltpu.sync_copy(i1_v, i1_hbm.at[sl])
        pltpu.sync_copy(i2_v, i2_hbm.at[sl])
        pltpu.sync_copy(w1_v, w1_hbm.at[sl])
        pltpu.sync_copy(w2_v, w2_hbm.at[sl])
        pltpu.sync_copy(zsc_v, zsc_hbm.at[sl])


def _sched_kernel(i1_ref, i2_ref, sched_ref):
    i1 = i1_ref[...]
    i2 = i2_ref[...]
    eio = lax.broadcasted_iota(jnp.int32, (T, E), 1)
    hitf = ((i1 == eio) | (i2 == eio)).astype(jnp.float32)
    ones_col = jnp.ones((T, 1), jnp.float32)
    nhit = lax.dot_general(hitf, ones_col, (((0,), (0,)), ((), ())),
                           preferred_element_type=jnp.float32)
    activef = (nhit > 0.0).astype(jnp.float32)
    io0 = lax.broadcasted_iota(jnp.int32, (E, E), 0)
    io1 = lax.broadcasted_iota(jnp.int32, (E, E), 1)
    ltri = (io0 >= io1).astype(jnp.float32)
    rank = lax.dot_general(ltri, activef, (((1,), (0,)), ((), ())),
                           preferred_element_type=jnp.float32)
    jrow = lax.broadcasted_iota(jnp.int32, (1, E), 1)
    onehot = ((rank.astype(jnp.int32) == jrow + 1) &
              (activef > 0.0)).astype(jnp.float32)
    erow = lax.broadcasted_iota(jnp.int32, (1, E), 1).astype(jnp.float32)
    sched_row = lax.dot_general(erow, onehot, (((1,), (0,)), ((), ())),
                                preferred_element_type=jnp.float32)
    count = jnp.sum(activef, axis=0, keepdims=True).astype(jnp.int32)
    ecol = lax.broadcasted_iota(jnp.int32, (E, 1), 0)
    last = jnp.max(jnp.where(activef > 0.0, ecol, -1), axis=0, keepdims=True)
    sched = jnp.where(jrow < count, sched_row.astype(jnp.int32),
                      jnp.maximum(last, 0))
    sched_ref[...] = jnp.zeros((8, 128), jnp.int32)
    sched_ref[0:1, 0:E] = sched
    sched_ref[0:1, E:E + 1] = count


def _expert_kernel(sched_ref, count_ref, x_ref, i1_ref, i2_ref, w1_ref,
                   w2c_ref, zsc_ref, w13_ref, w2_ref, out_ref):
    i = pl.program_id(0)
    e = sched_ref[i]

    @pl.when(i == 0)
    def _init():
        out_ref[...] = x_ref[...] * zsc_ref[...]

    @pl.when(i < count_ref[0])
    def _expert():
        x = x_ref[...]
        h13 = lax.dot_general(x, w13_ref[0], (((1,), (1,)), ((), ())),
                              preferred_element_type=jnp.float32)
        gate = h13[:, :F]
        up = h13[:, F:]
        act = gate * jax.nn.sigmoid(gate) * up
        oute = lax.dot_general(act, w2_ref[0], (((1,), (1,)), ((), ())),
                               preferred_element_type=jnp.float32)
        wcol = (jnp.where(i1_ref[...] == e, w1_ref[...], 0.0)
                + jnp.where(i2_ref[...] == e, w2c_ref[...], 0.0))
        out_ref[...] += wcol * oute


def kernel(hidden_states, router_weight, e_score_correction_bias, w13_weight, w2_weight):
    biasT = e_score_correction_bias.reshape(NC, 1)
    sT, scT = pl.pallas_call(
        _logits_kernel,
        in_specs=[
            pl.BlockSpec((T, H), lambda: (0, 0)),
            pl.BlockSpec((NC, H), lambda: (0, 0)),
            pl.BlockSpec((NC, 1), lambda: (0, 0)),
        ],
        out_specs=[
            pl.BlockSpec((NC, T), lambda: (0, 0)),
            pl.BlockSpec((NC, T), lambda: (0, 0)),
        ],
        out_shape=[
            jax.ShapeDtypeStruct((NC, T), jnp.float32),
            jax.ShapeDtypeStruct((NC, T), jnp.float32),
        ],
    )(hidden_states, router_weight, biasT)

    sc_router = functools.partial(
        pl.kernel,
        out_type=[
            jax.ShapeDtypeStruct((T,), jnp.int32),
            jax.ShapeDtypeStruct((T,), jnp.int32),
            jax.ShapeDtypeStruct((T,), jnp.float32),
            jax.ShapeDtypeStruct((T,), jnp.float32),
            jax.ShapeDtypeStruct((T,), jnp.float32),
        ],
        mesh=plsc.VectorSubcoreMesh(core_axis_name="c", subcore_axis_name="s"),
        scratch_types=[
            pltpu.VMEM((NC, T), jnp.float32),
            pltpu.VMEM((NC, T), jnp.float32),
            pltpu.VMEM((16,), jnp.int32),
            pltpu.VMEM((16,), jnp.int32),
            pltpu.VMEM((16,), jnp.float32),
            pltpu.VMEM((16,), jnp.float32),
            pltpu.VMEM((16,), jnp.float32),
        ],
    )(_sc_router)
    i1, i2, w1, w2, zsc = sc_router(sT, scT)

    i1c = i1.reshape(T, 1)
    i2c = i2.reshape(T, 1)
    w1c = w1.reshape(T, 1)
    w2c = w2.reshape(T, 1)
    zscc = zsc.reshape(T, 1)

    sched2d = pl.pallas_call(
        _sched_kernel,
        in_specs=[
            pl.BlockSpec((T, 1), lambda: (0, 0)),
            pl.BlockSpec((T, 1), lambda: (0, 0)),
        ],
        out_specs=pl.BlockSpec((8, 128), lambda: (0, 0)),
        out_shape=jax.ShapeDtypeStruct((8, 128), jnp.int32),
    )(i1c, i2c)

    sched = sched2d[0, 0:E]
    count = sched2d[0, E:E + 1]

    grid_spec = pltpu.PrefetchScalarGridSpec(
        num_scalar_prefetch=2,
        grid=(E,),
        in_specs=[
            pl.BlockSpec((T, H), lambda i, s, c: (0, 0)),
            pl.BlockSpec((T, 1), lambda i, s, c: (0, 0)),
            pl.BlockSpec((T, 1), lambda i, s, c: (0, 0)),
            pl.BlockSpec((T, 1), lambda i, s, c: (0, 0)),
            pl.BlockSpec((T, 1), lambda i, s, c: (0, 0)),
            pl.BlockSpec((T, 1), lambda i, s, c: (0, 0)),
            pl.BlockSpec((1, 2 * F, H), lambda i, s, c: (s[i], 0, 0)),
            pl.BlockSpec((1, H, F), lambda i, s, c: (s[i], 0, 0)),
        ],
        out_specs=pl.BlockSpec((T, H), lambda i, s, c: (0, 0)),
    )
    return pl.pallas_call(
        _expert_kernel,
        grid_spec=grid_spec,
        out_shape=jax.ShapeDtypeStruct((T, H), jnp.float32),
    )(sched, count, hidden_states, i1c, i2c, w1c, w2c, zscc, w13_weight, w2_weight)
